# Initial kernel scaffold; baseline (speedup 1.0000x reference)
import jax
import jax.numpy as jnp
from jax.experimental import pallas as pl


def kernel(query, W1, b1, W2, b2, case_embeddings, k):
    raise NotImplementedError("write your pallas kernel here")


# speedup vs baseline: 3.2632x; 3.2632x over previous
import functools

import jax
import jax.numpy as jnp
from jax import lax
from jax.experimental import pallas as pl
from jax.experimental.pallas import tpu as pltpu
from jax.experimental.pallas import tpu_sc as plsc

HIDDEN = 4096
EMBED = 256
Q = 1024
MAX_CASES = 100000

CB = 2048
NBLK = 49
CASES_PAD = NBLK * CB
CHUNK = 128
CPB = CB // CHUNK
NCHUNK = NBLK * CPB
NSEL = 16
KTOP = 10

KBLK = 256

NC_SC = 2
NS_SC = 16
NW = NC_SC * NS_SC
QPW = Q // NW
GRP = 8
NGRP = QPW // GRP


def _mlp_body(q_ref, w1_ref, b1_ref, w2_ref, b2_ref, qn_ref, acc_ref):
    kk = pl.program_id(0)

    @pl.when(kk == 0)
    def _():
        acc_ref[...] = jnp.zeros_like(acc_ref)

    acc_ref[...] += lax.dot_general(
        q_ref[...], w1_ref[...], (((1,), (1,)), ((), ())),
        preferred_element_type=jnp.float32)

    @pl.when(kk == pl.num_programs(0) - 1)
    def _():
        h = jnp.maximum(acc_ref[...] + b1_ref[...], 0.0)
        qe = lax.dot_general(
            h, w2_ref[...], (((1,), (1,)), ((), ())),
            preferred_element_type=jnp.float32) + b2_ref[...]
        nrm = jnp.sqrt(jnp.sum(qe * qe, axis=1, keepdims=True))
        qn_ref[...] = qe / (nrm + 1e-8)


def _encode(query, W1, b1, W2, b2, interpret=False):
    nk = HIDDEN // KBLK
    return pl.pallas_call(
        _mlp_body,
        grid=(nk,),
        in_specs=[
            pl.BlockSpec((Q, KBLK), lambda k: (0, k)),
            pl.BlockSpec((HIDDEN, KBLK), lambda k: (0, k)),
            pl.BlockSpec((1, HIDDEN), lambda k: (0, 0)),
            pl.BlockSpec((EMBED, HIDDEN), lambda k: (0, 0)),
            pl.BlockSpec((1, EMBED), lambda k: (0, 0)),
        ],
        out_specs=pl.BlockSpec((Q, EMBED), lambda k: (0, 0)),
        out_shape=jax.ShapeDtypeStruct((Q, EMBED), jnp.float32),
        scratch_shapes=[pltpu.VMEM((Q, HIDDEN), jnp.float32)],
        interpret=interpret,
    )(query, W1, b1.reshape(1, HIDDEN), W2, b2.reshape(1, EMBED))


def _sims_body(qn_ref, ce_ref, sims_ref, ids_ref, cmax_ref):
    j = pl.program_id(0)
    ce = ce_ref[...]
    nrm = jnp.sqrt(jnp.sum(ce * ce, axis=1, keepdims=True))
    cn = ce / (nrm + 1e-8)
    s = lax.dot_general(
        qn_ref[...], cn, (((1,), (1,)), ((), ())),
        preferred_element_type=jnp.float32)
    col = lax.broadcasted_iota(jnp.int32, (Q, CB), 1) + j * CB
    s = jnp.where(col < MAX_CASES, s, -2.0)
    sims_ref[...] = s
    for c in range(CPB):
        m = jnp.max(s[:, c * CHUNK:(c + 1) * CHUNK], axis=1)
        cmax_ref[pl.ds(j * CPB + c, 1), :] = m[None, :]

    @pl.when(j == pl.num_programs(0) - 1)
    def _():
        sm = cmax_ref[...]
        rows = lax.broadcasted_iota(jnp.int32, (NCHUNK, Q), 0)
        for t in range(NSEL):
            mx = jnp.max(sm, axis=0)
            cand = jnp.where(sm == mx[None, :], rows, jnp.int32(1 << 30))
            sel = jnp.min(cand, axis=0)
            ids_ref[pl.ds(t, 1), :] = sel[None, :]
            sm = jnp.where(rows == sel[None, :], jnp.float32(-3.0), sm)


def _sims(qn, case_embeddings, interpret=False):
    return pl.pallas_call(
        _sims_body,
        grid=(NBLK,),
        in_specs=[
            pl.BlockSpec((Q, EMBED), lambda j: (0, 0)),
            pl.BlockSpec((CB, EMBED), lambda j: (j, 0)),
        ],
        out_specs=[
            pl.BlockSpec((Q, CB), lambda j: (0, j)),
            pl.BlockSpec((NSEL, Q), lambda j: (0, 0)),
        ],
        out_shape=[
            jax.ShapeDtypeStruct((Q, CASES_PAD), jnp.float32),
            jax.ShapeDtypeStruct((NSEL, Q), jnp.int32),
        ],
        scratch_shapes=[pltpu.VMEM((NCHUNK, Q), jnp.float32)],
        interpret=interpret,
    )(qn, case_embeddings)


def _sc_gather(sims_flat, ids_q):
    mesh = plsc.VectorSubcoreMesh(core_axis_name="c", subcore_axis_name="s")

    @functools.partial(
        pl.kernel,
        mesh=mesh,
        out_type=jax.ShapeDtypeStruct((Q * NSEL, CHUNK), jnp.float32),
        scratch_types=[
            pltpu.VMEM((GRP, NSEL), jnp.int32),
            pltpu.VMEM((GRP * NSEL,), jnp.int32),
            pltpu.VMEM((GRP * NSEL, CHUNK), jnp.float32),
            pltpu.SemaphoreType.DMA,
        ],
    )
    def k(sims_hbm, ids_hbm, out_hbm, ids_v, idx_v, rows_v, sem):
        wid = lax.axis_index("s") * NC_SC + lax.axis_index("c")
        for g in range(NGRP):
            qbase = wid * QPW + g * GRP
            pltpu.sync_copy(ids_hbm.at[pl.ds(qbase, GRP)], ids_v)
            for i in range(GRP):
                idx_v[pl.ds(i * NSEL, NSEL)] = (
                    ids_v[i] + (qbase + i) * NCHUNK)
            pltpu.async_copy(sims_hbm.at[idx_v], rows_v, sem).wait()
            pltpu.sync_copy(rows_v, out_hbm.at[pl.ds(qbase * NSEL, GRP * NSEL)])

    return k(sims_flat, ids_q)


def _final_body(vals_ref, ids_ref, tv_ref, ti_ref):
    v = vals_ref[...]
    idsq = ids_ref[...]
    off = lax.broadcasted_iota(jnp.int32, (Q, CHUNK), 1)
    g = jnp.concatenate(
        [idsq[:, jj:jj + 1] * CHUNK + off for jj in range(NSEL)], axis=1)
    big = jnp.int32(1 << 30)
    for t in range(KTOP):
        m = jnp.max(v, axis=1)
        gm = jnp.min(jnp.where(v == m[:, None], g, big), axis=1)
        tv_ref[:, pl.ds(t, 1)] = m[:, None]
        ti_ref[:, pl.ds(t, 1)] = gm[:, None]
        v = jnp.where((v == m[:, None]) & (g == gm[:, None]),
                      jnp.float32(-4.0), v)


def _final(cand_vals, ids_q, interpret=False):
    return pl.pallas_call(
        _final_body,
        out_shape=[
            jax.ShapeDtypeStruct((Q, 16), jnp.float32),
            jax.ShapeDtypeStruct((Q, 16), jnp.int32),
        ],
        interpret=interpret,
    )(cand_vals, ids_q)


def kernel(query, W1, b1, W2, b2, case_embeddings, k):
    qn = _encode(query, W1, b1, W2, b2)
    sims, ids = _sims(qn, case_embeddings)
    sims_flat = sims.reshape(Q * NCHUNK, CHUNK)
    ids_q = ids.T
    cand = _sc_gather(sims_flat, ids_q)
    cand_vals = cand.reshape(Q, NSEL * CHUNK)
    tv, ti = _final(cand_vals, ids_q)
    return tv[:, :KTOP], ti[:, :KTOP]


# speedup vs baseline: 4.9773x; 1.5253x over previous
import functools

import jax
import jax.numpy as jnp
from jax import lax
from jax.experimental import pallas as pl
from jax.experimental.pallas import tpu as pltpu
from jax.experimental.pallas import tpu_sc as plsc

HIDDEN = 4096
EMBED = 256
Q = 1024
MAX_CASES = 100000

CB = 2048
NBLK = 49
CASES_PAD = NBLK * CB
CHUNK = 128
CPB = CB // CHUNK
NCHUNK = NBLK * CPB
NSEL = 16
KTOP = 10

KBLK = 256

NC_SC = 2
NS_SC = 16
NW = NC_SC * NS_SC
QPW = Q // NW
GRP = 8
NGRP = QPW // GRP


def _mlp_body(q_ref, w1_ref, b1_ref, w2_ref, b2_ref, qn_ref, acc_ref):
    kk = pl.program_id(0)

    @pl.when(kk == 0)
    def _():
        acc_ref[...] = jnp.zeros_like(acc_ref)

    acc_ref[...] += lax.dot_general(
        q_ref[...], w1_ref[...], (((1,), (1,)), ((), ())),
        preferred_element_type=jnp.float32)

    @pl.when(kk == pl.num_programs(0) - 1)
    def _():
        h = jnp.maximum(acc_ref[...] + b1_ref[...], 0.0)
        qe = lax.dot_general(
            h, w2_ref[...], (((1,), (1,)), ((), ())),
            preferred_element_type=jnp.float32) + b2_ref[...]
        nrm = jnp.sqrt(jnp.sum(qe * qe, axis=1, keepdims=True))
        qn_ref[...] = qe / (nrm + 1e-8)


def _encode(query, W1, b1, W2, b2, interpret=False):
    nk = HIDDEN // KBLK
    return pl.pallas_call(
        _mlp_body,
        grid=(nk,),
        in_specs=[
            pl.BlockSpec((Q, KBLK), lambda k: (0, k)),
            pl.BlockSpec((HIDDEN, KBLK), lambda k: (0, k)),
            pl.BlockSpec((1, HIDDEN), lambda k: (0, 0)),
            pl.BlockSpec((EMBED, HIDDEN), lambda k: (0, 0)),
            pl.BlockSpec((1, EMBED), lambda k: (0, 0)),
        ],
        out_specs=pl.BlockSpec((Q, EMBED), lambda k: (0, 0)),
        out_shape=jax.ShapeDtypeStruct((Q, EMBED), jnp.float32),
        scratch_shapes=[pltpu.VMEM((Q, HIDDEN), jnp.float32)],
        interpret=interpret,
    )(query, W1, b1.reshape(1, HIDDEN), W2, b2.reshape(1, EMBED))


def _sims_body(qn_ref, ce_ref, sims_ref, ids_ref, cmax_ref):
    j = pl.program_id(0)
    ce = ce_ref[...]
    nrm = jnp.sqrt(jnp.sum(ce * ce, axis=1, keepdims=True))
    cn = ce / (nrm + 1e-8)
    s = lax.dot_general(
        qn_ref[...], cn, (((1,), (1,)), ((), ())),
        preferred_element_type=jnp.float32)
    col = lax.broadcasted_iota(jnp.int32, (Q, CB), 1) + j * CB
    s = jnp.where(col < MAX_CASES, s, -2.0)
    for c in range(CPB):
        blk = s[:, c * CHUNK:(c + 1) * CHUNK]
        sims_ref[c * Q:(c + 1) * Q, :] = blk
        m = jnp.max(blk, axis=1)
        cmax_ref[pl.ds(j * CPB + c, 1), :] = m[None, :]

    @pl.when(j == pl.num_programs(0) - 1)
    def _():
        sm = cmax_ref[...]
        rows = lax.broadcasted_iota(jnp.int32, (NCHUNK, Q), 0)
        for t in range(NSEL):
            mx = jnp.max(sm, axis=0)
            cand = jnp.where(sm == mx[None, :], rows, jnp.int32(1 << 30))
            sel = jnp.min(cand, axis=0)
            ids_ref[pl.ds(t, 1), :] = sel[None, :]
            sm = jnp.where(rows == sel[None, :], jnp.float32(-3.0), sm)


def _sims(qn, case_embeddings, interpret=False):
    return pl.pallas_call(
        _sims_body,
        grid=(NBLK,),
        in_specs=[
            pl.BlockSpec((Q, EMBED), lambda j: (0, 0)),
            pl.BlockSpec((CB, EMBED), lambda j: (j, 0)),
        ],
        out_specs=[
            pl.BlockSpec((CPB * Q, CHUNK), lambda j: (j, 0)),
            pl.BlockSpec((NSEL, Q), lambda j: (0, 0)),
        ],
        out_shape=[
            jax.ShapeDtypeStruct((NCHUNK * Q, CHUNK), jnp.float32),
            jax.ShapeDtypeStruct((NSEL, Q), jnp.int32),
        ],
        scratch_shapes=[pltpu.VMEM((NCHUNK, Q), jnp.float32)],
        interpret=interpret,
    )(qn, case_embeddings)


def _sc_gather(sims_flat, ids_q):
    mesh = plsc.VectorSubcoreMesh(core_axis_name="c", subcore_axis_name="s")

    @functools.partial(
        pl.kernel,
        mesh=mesh,
        out_type=jax.ShapeDtypeStruct((Q * NSEL, CHUNK), jnp.float32),
        scratch_types=[
            pltpu.VMEM((GRP, NSEL), jnp.int32),
            pltpu.VMEM((GRP * NSEL,), jnp.int32),
            pltpu.VMEM((GRP * NSEL, CHUNK), jnp.float32),
            pltpu.SemaphoreType.DMA,
        ],
    )
    def k(sims_hbm, ids_hbm, out_hbm, ids_v, idx_v, rows_v, sem):
        wid = lax.axis_index("s") * NC_SC + lax.axis_index("c")
        for g in range(NGRP):
            qbase = wid * QPW + g * GRP
            pltpu.sync_copy(ids_hbm.at[pl.ds(qbase, GRP)], ids_v)
            for i in range(GRP):
                idx_v[pl.ds(i * NSEL, NSEL)] = ids_v[i] * Q + (qbase + i)
            pltpu.async_copy(sims_hbm.at[idx_v], rows_v, sem).wait()
            pltpu.sync_copy(rows_v, out_hbm.at[pl.ds(qbase * NSEL, GRP * NSEL)])

    return k(sims_flat, ids_q)


def _final_body(vals_ref, ids_ref, tv_ref, ti_ref):
    v = vals_ref[...]
    idsq = ids_ref[...]
    off = lax.broadcasted_iota(jnp.int32, (Q, CHUNK), 1)
    g = jnp.concatenate(
        [idsq[:, jj:jj + 1] * CHUNK + off for jj in range(NSEL)], axis=1)
    big = jnp.int32(1 << 30)
    for t in range(KTOP):
        m = jnp.max(v, axis=1)
        gm = jnp.min(jnp.where(v == m[:, None], g, big), axis=1)
        tv_ref[:, pl.ds(t, 1)] = m[:, None]
        ti_ref[:, pl.ds(t, 1)] = gm[:, None]
        v = jnp.where((v == m[:, None]) & (g == gm[:, None]),
                      jnp.float32(-4.0), v)


def _final(cand_vals, ids_q, interpret=False):
    return pl.pallas_call(
        _final_body,
        out_shape=[
            jax.ShapeDtypeStruct((Q, 16), jnp.float32),
            jax.ShapeDtypeStruct((Q, 16), jnp.int32),
        ],
        interpret=interpret,
    )(cand_vals, ids_q)


def kernel(query, W1, b1, W2, b2, case_embeddings, k):
    qn = _encode(query, W1, b1, W2, b2)
    sims_flat, ids = _sims(qn, case_embeddings)
    ids_q = ids.T
    cand = _sc_gather(sims_flat, ids_q)
    cand_vals = cand.reshape(Q, NSEL * CHUNK)
    tv, ti = _final(cand_vals, ids_q)
    return tv[:, :KTOP], ti[:, :KTOP]


# speedup vs baseline: 5.1183x; 1.0283x over previous
import functools

import jax
import jax.numpy as jnp
from jax import lax
from jax.experimental import pallas as pl
from jax.experimental.pallas import tpu as pltpu
from jax.experimental.pallas import tpu_sc as plsc

HIDDEN = 4096
EMBED = 256
Q = 1024
MAX_CASES = 100000

CB = 2048
NBLK = 49
CASES_PAD = NBLK * CB
CHUNK = 128
CPB = CB // CHUNK
NCHUNK = NBLK * CPB
NSEL = 16
KTOP = 10

KBLK = 256

NC_SC = 2
NS_SC = 16
NW = NC_SC * NS_SC
QPW = Q // NW
GRP = 8
NGRP = QPW // GRP


def _mlp_body(q_ref, w1_ref, b1_ref, w2_ref, b2_ref, qn_ref, acc_ref):
    kk = pl.program_id(0)

    @pl.when(kk == 0)
    def _():
        acc_ref[...] = jnp.zeros_like(acc_ref)

    acc_ref[...] += lax.dot_general(
        q_ref[...], w1_ref[...], (((1,), (1,)), ((), ())),
        preferred_element_type=jnp.float32)

    @pl.when(kk == pl.num_programs(0) - 1)
    def _():
        h = jnp.maximum(acc_ref[...] + b1_ref[...], 0.0)
        qe = lax.dot_general(
            h, w2_ref[...], (((1,), (1,)), ((), ())),
            preferred_element_type=jnp.float32) + b2_ref[...]
        nrm = jnp.sqrt(jnp.sum(qe * qe, axis=1, keepdims=True))
        qn_ref[...] = qe / (nrm + 1e-8)


def _encode(query, W1, b1, W2, b2, interpret=False):
    nk = HIDDEN // KBLK
    return pl.pallas_call(
        _mlp_body,
        grid=(nk,),
        in_specs=[
            pl.BlockSpec((Q, KBLK), lambda k: (0, k)),
            pl.BlockSpec((HIDDEN, KBLK), lambda k: (0, k)),
            pl.BlockSpec((1, HIDDEN), lambda k: (0, 0)),
            pl.BlockSpec((EMBED, HIDDEN), lambda k: (0, 0)),
            pl.BlockSpec((1, EMBED), lambda k: (0, 0)),
        ],
        out_specs=pl.BlockSpec((Q, EMBED), lambda k: (0, 0)),
        out_shape=jax.ShapeDtypeStruct((Q, EMBED), jnp.float32),
        scratch_shapes=[pltpu.VMEM((Q, HIDDEN), jnp.float32)],
        interpret=interpret,
    )(query, W1, b1.reshape(1, HIDDEN), W2, b2.reshape(1, EMBED))


def _sims_body(qn_ref, ce_ref, sims_ref, ids_ref, cmax_ref):
    j = pl.program_id(0)
    ce = ce_ref[...]
    nrm = jnp.sqrt(jnp.sum(ce * ce, axis=1, keepdims=True))
    cn = ce / (nrm + 1e-8)
    s = lax.dot_general(
        qn_ref[...], cn, (((1,), (1,)), ((), ())),
        preferred_element_type=jnp.float32)
    for c in range(CPB):
        blk = s[:, c * CHUNK:(c + 1) * CHUNK]
        sims_ref[c * Q:(c + 1) * Q, :] = blk
        m = jnp.max(blk, axis=1)
        cmax_ref[pl.ds(j * CPB + c, 1), :] = m[None, :]

    @pl.when(j == pl.num_programs(0) - 1)
    def _():
        first_pad_chunk = (MAX_CASES - (NBLK - 1) * CB) // CHUNK
        for c in range(first_pad_chunk, CPB):
            col = (lax.broadcasted_iota(jnp.int32, (Q, CHUNK), 1)
                   + (NBLK - 1) * CB + c * CHUNK)
            blk = jnp.where(col < MAX_CASES,
                            s[:, c * CHUNK:(c + 1) * CHUNK],
                            jnp.float32(-2.0))
            sims_ref[c * Q:(c + 1) * Q, :] = blk
            m = jnp.max(blk, axis=1)
            cmax_ref[pl.ds((NBLK - 1) * CPB + c, 1), :] = m[None, :]

        sm = cmax_ref[...]
        rows = lax.broadcasted_iota(jnp.int32, (NCHUNK, Q), 0)
        for t in range(NSEL):
            mx = jnp.max(sm, axis=0)
            cand = jnp.where(sm == mx[None, :], rows, jnp.int32(1 << 30))
            sel = jnp.min(cand, axis=0)
            ids_ref[pl.ds(t, 1), :] = sel[None, :]
            sm = jnp.where(rows == sel[None, :], jnp.float32(-3.0), sm)


def _sims(qn, case_embeddings, interpret=False):
    return pl.pallas_call(
        _sims_body,
        grid=(NBLK,),
        in_specs=[
            pl.BlockSpec((Q, EMBED), lambda j: (0, 0)),
            pl.BlockSpec((CB, EMBED), lambda j: (j, 0)),
        ],
        out_specs=[
            pl.BlockSpec((CPB * Q, CHUNK), lambda j: (j, 0)),
            pl.BlockSpec((NSEL, Q), lambda j: (0, 0)),
        ],
        out_shape=[
            jax.ShapeDtypeStruct((NCHUNK * Q, CHUNK), jnp.float32),
            jax.ShapeDtypeStruct((NSEL, Q), jnp.int32),
        ],
        scratch_shapes=[pltpu.VMEM((NCHUNK, Q), jnp.float32)],
        interpret=interpret,
    )(qn, case_embeddings)


def _sc_gather(sims_flat, ids_q):
    mesh = plsc.VectorSubcoreMesh(core_axis_name="c", subcore_axis_name="s")

    @functools.partial(
        pl.kernel,
        mesh=mesh,
        out_type=jax.ShapeDtypeStruct((Q * NSEL, CHUNK), jnp.float32),
        scratch_types=[
            pltpu.VMEM((GRP, NSEL), jnp.int32),
            pltpu.VMEM((GRP * NSEL,), jnp.int32),
            pltpu.VMEM((GRP * NSEL, CHUNK), jnp.float32),
            pltpu.SemaphoreType.DMA,
        ],
    )
    def k(sims_hbm, ids_hbm, out_hbm, ids_v, idx_v, rows_v, sem):
        wid = lax.axis_index("s") * NC_SC + lax.axis_index("c")
        for g in range(NGRP):
            qbase = wid * QPW + g * GRP
            pltpu.sync_copy(ids_hbm.at[pl.ds(qbase, GRP)], ids_v)
            for i in range(GRP):
                idx_v[pl.ds(i * NSEL, NSEL)] = ids_v[i] * Q + (qbase + i)
            pltpu.async_copy(sims_hbm.at[idx_v], rows_v, sem).wait()
            pltpu.sync_copy(rows_v, out_hbm.at[pl.ds(qbase * NSEL, GRP * NSEL)])

    return k(sims_flat, ids_q)


def _final_body(vals_ref, ids_ref, tv_ref, ti_ref):
    v = vals_ref[...]
    idsq = ids_ref[...]
    off = lax.broadcasted_iota(jnp.int32, (Q, CHUNK), 1)
    g = jnp.concatenate(
        [idsq[:, jj:jj + 1] * CHUNK + off for jj in range(NSEL)], axis=1)
    big = jnp.int32(1 << 30)
    for t in range(KTOP):
        m = jnp.max(v, axis=1)
        gm = jnp.min(jnp.where(v == m[:, None], g, big), axis=1)
        tv_ref[:, pl.ds(t, 1)] = m[:, None]
        ti_ref[:, pl.ds(t, 1)] = gm[:, None]
        v = jnp.where((v == m[:, None]) & (g == gm[:, None]),
                      jnp.float32(-4.0), v)


def _final(cand_vals, ids_q, interpret=False):
    return pl.pallas_call(
        _final_body,
        out_shape=[
            jax.ShapeDtypeStruct((Q, 16), jnp.float32),
            jax.ShapeDtypeStruct((Q, 16), jnp.int32),
        ],
        interpret=interpret,
    )(cand_vals, ids_q)


def kernel(query, W1, b1, W2, b2, case_embeddings, k):
    qn = _encode(query, W1, b1, W2, b2)
    sims_flat, ids = _sims(qn, case_embeddings)
    ids_q = ids.T
    cand = _sc_gather(sims_flat, ids_q)
    cand_vals = cand.reshape(Q, NSEL * CHUNK)
    tv, ti = _final(cand_vals, ids_q)
    return tv[:, :KTOP], ti[:, :KTOP]


# speedup vs baseline: 6.9233x; 1.3527x over previous
import functools

import jax
import jax.numpy as jnp
from jax import lax
from jax.experimental import pallas as pl
from jax.experimental.pallas import tpu as pltpu
from jax.experimental.pallas import tpu_sc as plsc

HIDDEN = 4096
EMBED = 256
Q = 1024
MAX_CASES = 100000

CB = 2048
NBLK = 49
CASES_PAD = NBLK * CB
CHUNK = 128
CPB = CB // CHUNK
NCHUNK = NBLK * CPB
NSEL = 16
KTOP = 10

KBLK = 256

NC_SC = 2
NS_SC = 16
NW = NC_SC * NS_SC
QPW = Q // NW
GRP = 8
NGRP = QPW // GRP


def _mlp1_body(q_ref, w1_ref, b1_ref, h_ref):
    h_ref[...] = jnp.maximum(
        lax.dot_general(q_ref[...], w1_ref[...], (((1,), (1,)), ((), ())),
                        preferred_element_type=jnp.float32) + b1_ref[...],
        0.0)


def _mlp2_body(h_ref, w2_ref, b2_ref, qn_ref):
    qe = lax.dot_general(
        h_ref[...], w2_ref[...], (((1,), (1,)), ((), ())),
        preferred_element_type=jnp.float32) + b2_ref[...]
    nrm = jnp.sqrt(jnp.sum(qe * qe, axis=1, keepdims=True))
    qn_ref[...] = qe / (nrm + 1e-8)


JB = 512


def _encode(query, W1, b1, W2, b2, interpret=False):
    h = pl.pallas_call(
        _mlp1_body,
        grid=(HIDDEN // JB,),
        in_specs=[
            pl.BlockSpec((Q, HIDDEN), lambda j: (0, 0)),
            pl.BlockSpec((JB, HIDDEN), lambda j: (j, 0)),
            pl.BlockSpec((1, JB), lambda j: (0, j)),
        ],
        out_specs=pl.BlockSpec((Q, JB), lambda j: (0, j)),
        out_shape=jax.ShapeDtypeStruct((Q, HIDDEN), jnp.float32),
        interpret=interpret,
    )(query, W1, b1.reshape(1, HIDDEN))
    return pl.pallas_call(
        _mlp2_body,
        in_specs=[
            pl.BlockSpec((Q, HIDDEN), lambda: (0, 0)),
            pl.BlockSpec((EMBED, HIDDEN), lambda: (0, 0)),
            pl.BlockSpec((1, EMBED), lambda: (0, 0)),
        ],
        out_specs=pl.BlockSpec((Q, EMBED), lambda: (0, 0)),
        out_shape=jax.ShapeDtypeStruct((Q, EMBED), jnp.float32),
        interpret=interpret,
    )(h, W2, b2.reshape(1, EMBED))


def _sims_body(qn_ref, ce_ref, sims_ref, cmax_ref):
    j = pl.program_id(0)
    ce = ce_ref[...]
    nrm = jnp.sqrt(jnp.sum(ce * ce, axis=1, keepdims=True))
    cn = ce / (nrm + 1e-8)
    s = lax.dot_general(
        qn_ref[...], cn, (((1,), (1,)), ((), ())),
        preferred_element_type=jnp.float32)
    for c in range(CPB):
        blk = s[:, c * CHUNK:(c + 1) * CHUNK]
        sims_ref[c * Q:(c + 1) * Q, :] = blk
    bm = jnp.max(s.reshape(Q, CPB, CHUNK), axis=2)
    cmax_ref[...] = bm[None]

    @pl.when(j == pl.num_programs(0) - 1)
    def _():
        first_pad_chunk = (MAX_CASES - (NBLK - 1) * CB) // CHUNK
        for c in range(first_pad_chunk, CPB):
            col = (lax.broadcasted_iota(jnp.int32, (Q, CHUNK), 1)
                   + (NBLK - 1) * CB + c * CHUNK)
            blk = jnp.where(col < MAX_CASES,
                            s[:, c * CHUNK:(c + 1) * CHUNK],
                            jnp.float32(-2.0))
            sims_ref[c * Q:(c + 1) * Q, :] = blk
            m = jnp.max(blk, axis=1)
            cmax_ref[0, :, c:c + 1] = m[:, None]


def _sims(qn, case_embeddings, interpret=False):
    return pl.pallas_call(
        _sims_body,
        grid=(NBLK,),
        in_specs=[
            pl.BlockSpec((Q, EMBED), lambda j: (0, 0)),
            pl.BlockSpec((CB, EMBED), lambda j: (j, 0)),
        ],
        out_specs=[
            pl.BlockSpec((CPB * Q, CHUNK), lambda j: (j, 0)),
            pl.BlockSpec((1, Q, CPB), lambda j: (j, 0, 0)),
        ],
        out_shape=[
            jax.ShapeDtypeStruct((NCHUNK * Q, CHUNK), jnp.float32),
            jax.ShapeDtypeStruct((NBLK, Q, CPB), jnp.float32),
        ],
        interpret=interpret,
    )(qn, case_embeddings)


def _select_body(cmax_ref, ids_ref):
    sm = jnp.concatenate(
        [cmax_ref[jj] for jj in range(NBLK)], axis=1)
    cols = lax.broadcasted_iota(jnp.int32, (Q, NCHUNK), 1)
    for t in range(NSEL):
        mx = jnp.max(sm, axis=1)
        cand = jnp.where(sm == mx[:, None], cols, jnp.int32(1 << 30))
        sel = jnp.min(cand, axis=1)
        ids_ref[:, pl.ds(t, 1)] = sel[:, None]
        sm = jnp.where(cols == sel[:, None], jnp.float32(-3.0), sm)


def _select(cmax, interpret=False):
    return pl.pallas_call(
        _select_body,
        out_shape=jax.ShapeDtypeStruct((Q, NSEL), jnp.int32),
        interpret=interpret,
    )(cmax)


def _sc_gather(sims_flat, ids_q):
    mesh = plsc.VectorSubcoreMesh(core_axis_name="c", subcore_axis_name="s")

    @functools.partial(
        pl.kernel,
        mesh=mesh,
        out_type=jax.ShapeDtypeStruct((Q * NSEL, CHUNK), jnp.float32),
        scratch_types=[
            pltpu.VMEM((GRP, NSEL), jnp.int32),
            pltpu.VMEM((GRP * NSEL,), jnp.int32),
            pltpu.VMEM((GRP * NSEL, CHUNK), jnp.float32),
            pltpu.SemaphoreType.DMA,
        ],
    )
    def k(sims_hbm, ids_hbm, out_hbm, ids_v, idx_v, rows_v, sem):
        wid = lax.axis_index("s") * NC_SC + lax.axis_index("c")
        for g in range(NGRP):
            qbase = wid * QPW + g * GRP
            pltpu.sync_copy(ids_hbm.at[pl.ds(qbase, GRP)], ids_v)
            for i in range(GRP):
                idx_v[pl.ds(i * NSEL, NSEL)] = ids_v[i] * Q + (qbase + i)
            pltpu.async_copy(sims_hbm.at[idx_v], rows_v, sem).wait()
            pltpu.sync_copy(rows_v, out_hbm.at[pl.ds(qbase * NSEL, GRP * NSEL)])

    return k(sims_flat, ids_q)


def _final_body(vals_ref, ids_ref, tv_ref, ti_ref):
    v = vals_ref[...]
    idsq = ids_ref[...]
    off = lax.broadcasted_iota(jnp.int32, (Q, CHUNK), 1)
    g = jnp.concatenate(
        [idsq[:, jj:jj + 1] * CHUNK + off for jj in range(NSEL)], axis=1)
    big = jnp.int32(1 << 30)
    for t in range(KTOP):
        m = jnp.max(v, axis=1)
        gm = jnp.min(jnp.where(v == m[:, None], g, big), axis=1)
        tv_ref[:, pl.ds(t, 1)] = m[:, None]
        ti_ref[:, pl.ds(t, 1)] = gm[:, None]
        v = jnp.where((v == m[:, None]) & (g == gm[:, None]),
                      jnp.float32(-4.0), v)


def _final(cand_vals, ids_q, interpret=False):
    return pl.pallas_call(
        _final_body,
        out_shape=[
            jax.ShapeDtypeStruct((Q, 16), jnp.float32),
            jax.ShapeDtypeStruct((Q, 16), jnp.int32),
        ],
        interpret=interpret,
    )(cand_vals, ids_q)


def kernel(query, W1, b1, W2, b2, case_embeddings, k):
    qn = _encode(query, W1, b1, W2, b2)
    sims_flat, cmax = _sims(qn, case_embeddings)
    ids_q = _select(cmax)
    cand = _sc_gather(sims_flat, ids_q)
    cand_vals = cand.reshape(Q, NSEL * CHUNK)
    tv, ti = _final(cand_vals, ids_q)
    return tv[:, :KTOP], ti[:, :KTOP]
